# baseline (speedup 1.0000x reference)
import functools

import jax
import jax.numpy as jnp
from jax import lax
from jax.experimental import pallas as pl
from jax.experimental.pallas import tpu as pltpu
from jax.experimental.pallas import tpu_sc as plsc

B = 16384
NCOL = 26
VOCAB = 100000
EDIM = 16
DIN = 200 + 1 + NCOL * EDIM

NC = 2
NS = 16
NW = NC * NS
B_TOT = B * NCOL
B_PER_W = B_TOT // NW
CHUNK = 1664
NCHUNK = B_PER_W // CHUNK


def _sc_gather(table_flat, idx_flat):
    mesh = plsc.VectorSubcoreMesh(core_axis_name="c", subcore_axis_name="s")

    @functools.partial(
        pl.kernel,
        mesh=mesh,
        out_type=jax.ShapeDtypeStruct((B_TOT, EDIM), jnp.float32),
        scratch_types=[
            pltpu.VMEM((CHUNK,), jnp.int32),
            pltpu.VMEM((CHUNK, EDIM), jnp.float32),
            pltpu.SemaphoreType.DMA,
        ],
        compiler_params=pltpu.CompilerParams(use_tc_tiling_on_sc=False),
    )
    def gather_k(table_hbm, idx_hbm, out_hbm, idx_v, rows_v, sem):
        wid = lax.axis_index("s") * NC + lax.axis_index("c")
        base = wid * B_PER_W
        for i in range(NCHUNK):
            off = base + i * CHUNK
            pltpu.sync_copy(idx_hbm.at[pl.ds(off, CHUNK)], idx_v)
            pltpu.async_copy(table_hbm.at[idx_v], rows_v, sem).wait()
            pltpu.sync_copy(rows_v, out_hbm.at[pl.ds(off, CHUNK)])

    return gather_k(table_flat, idx_flat)


BS = 512


def _mlp_body(meta_ref, nla_ref, emb_ref, w1m_ref, w1n_ref, w1e_ref,
              b1_ref, w2_ref, b2_ref, out_ref, pred_ref):
    m = meta_ref[...]
    n = nla_ref[...]
    e = emb_ref[...]
    out_ref[...] = jnp.concatenate([m, n, e], axis=1)
    h = jnp.dot(m, w1m_ref[...], preferred_element_type=jnp.float32)
    h = h + jnp.dot(e, w1e_ref[...], preferred_element_type=jnp.float32)
    h = h + n * w1n_ref[...]
    h = jnp.maximum(h + b1_ref[...], 0.0)
    z = jnp.dot(h, w2_ref[...], preferred_element_type=jnp.float32) + b2_ref[...]
    pred_ref[...] = 1.0 / (1.0 + jnp.exp(-z))


def _tc_mlp(meta, nla, emb, w1m, w1n, w1e, b1, w2, b2):
    grid = (B // BS,)
    blk = lambda r, c: pl.BlockSpec((r, c), lambda i: (i, 0))
    full = lambda r, c: pl.BlockSpec((r, c), lambda i: (0, 0))
    return pl.pallas_call(
        _mlp_body,
        grid=grid,
        in_specs=[
            blk(BS, 200), blk(BS, 1), blk(BS, NCOL * EDIM),
            full(200, 20), full(1, 20), full(NCOL * EDIM, 20),
            full(1, 20), full(20, 1), full(1, 1),
        ],
        out_specs=[blk(BS, DIN), blk(BS, 1)],
        out_shape=[
            jax.ShapeDtypeStruct((B, DIN), jnp.float32),
            jax.ShapeDtypeStruct((B, 1), jnp.float32),
        ],
    )(meta, nla, emb, w1m, w1n, w1e, b1, w2, b2)


def kernel(meta_features, nla, components, tables, W1, b1, W2, b2):
    table_flat = tables.reshape(NCOL * VOCAB, EDIM)
    col_off = (jnp.arange(NCOL, dtype=jnp.int32) * VOCAB)[None, :]
    idx_flat = (components.astype(jnp.int32) + col_off).reshape(B_TOT)

    emb_flat = _sc_gather(table_flat, idx_flat)
    emb = emb_flat.reshape(B, NCOL * EDIM)

    w1m = W1[0:200]
    w1n = W1[200:201]
    w1e = W1[201:DIN]
    embedding, pred = _tc_mlp(meta_features, nla, emb, w1m, w1n, w1e,
                              b1.reshape(1, 20), W2, b2.reshape(1, 1))
    return (embedding, pred)
